# baseline (speedup 1.0000x reference)
from functools import partial

import jax
import jax.numpy as jnp
from jax.experimental import pallas as pl
from jax.experimental.pallas import tpu as pltpu

V_J = 25
_BITS = (1, 2, 4, 8, 16)
_VMEM_LIMIT = 64 * 1024 * 1024


def _fused_kernel(x_ref, m_ref, s1_ref, b1_ref, w_ref, wt_ref, bt_ref,
                  sb_ref, bb_ref, o_ref, *, tm):
    c = x_ref.shape[-1]
    rows = V_J * tm
    xbf = x_ref[...]
    s_lane = jax.lax.broadcasted_iota(jnp.int32, (1, 1, c), 2) % V_J
    y = xbf
    for b in _BITS:
        y = jnp.where((s_lane & b) != 0,
                      jnp.concatenate([y[b:], y[:b]], axis=0), y)
    xs = y * m_ref[...]
    z = jnp.dot(xs.reshape(rows, c), w_ref[...],
                preferred_element_type=jnp.float32)
    t = (z.reshape(V_J, tm, c) * s1_ref[...] + b1_ref[...]).astype(jnp.bfloat16)
    for b in _BITS:
        t = jnp.where((s_lane & b) != 0,
                      jnp.concatenate([t[V_J - b:], t[:V_J - b]], axis=0), t)
    g = jnp.maximum(t + xbf, 0)
    u = jnp.dot(g.reshape(rows, c), wt_ref[...],
                preferred_element_type=jnp.float32)
    u = jnp.maximum(u + bt_ref[...], 0.0)
    out = jnp.maximum(u.reshape(V_J, tm, c) * sb_ref[...] + bb_ref[...] + xbf,
                      0.0)
    o_ref[...] = out


def _pick_tm(nt):
    tm = None
    d = 8
    while d <= min(nt, 256):
        if nt % d == 0:
            tm = d
        d += 8
    return nt if tm is None else tm


def kernel(x, linear_w, linear_b, feature_mask, bn1_scale, bn1_bias,
           bn_a_scale, bn_a_bias, bn_b_scale, bn_b_bias, wt, conv_b,
           idx_in, idx_out):
    n, c, t, v = x.shape
    assert v == V_J
    co = linear_w.shape[1]
    nt = n * t
    tm = _pick_tm(nt)

    mask = jnp.tanh(feature_mask) + 1.0
    b1 = bn1_bias + linear_b * bn1_scale
    wt_f = bn_a_scale.reshape(-1, 1) * wt
    bt_f = bn_a_bias @ wt + conv_b

    jj = jnp.arange(co)[None, :] % V_J
    s1p, b1p = bn1_scale, b1
    for b in _BITS:
        sel = (jj & b) != 0
        s1p = jnp.where(sel, jnp.roll(s1p, -b, axis=0), s1p)
        b1p = jnp.where(sel, jnp.roll(b1p, -b, axis=0), b1p)

    consts = (mask.astype(jnp.bfloat16).reshape(v, 1, c),
              s1p.reshape(v, 1, co), b1p.reshape(v, 1, co),
              linear_w.astype(jnp.bfloat16), wt_f.astype(jnp.bfloat16),
              bt_f, bn_b_scale.reshape(1, 1, co), bn_b_bias.reshape(1, 1, co))

    n_chunks = 4
    while n % n_chunks or (n // n_chunks) * t % tm:
        n_chunks //= 2
    nc = n // n_chunks
    ntc = nc * t

    def run_chunk(xc):
        xtc = jnp.transpose(xc, (3, 0, 2, 1)).reshape(v, ntc, c)
        xtc = xtc.astype(jnp.bfloat16)
        o = pl.pallas_call(
            partial(_fused_kernel, tm=tm),
            out_shape=jax.ShapeDtypeStruct((v, ntc, co), jnp.float32),
            grid=(ntc // tm,),
            in_specs=[
                pl.BlockSpec((v, tm, c), lambda i: (0, i, 0)),
                pl.BlockSpec((v, 1, c), lambda i: (0, 0, 0)),
                pl.BlockSpec((v, 1, co), lambda i: (0, 0, 0)),
                pl.BlockSpec((v, 1, co), lambda i: (0, 0, 0)),
                pl.BlockSpec((c, co), lambda i: (0, 0)),
                pl.BlockSpec((co, co), lambda i: (0, 0)),
                pl.BlockSpec((1, co), lambda i: (0, 0)),
                pl.BlockSpec((1, 1, co), lambda i: (0, 0, 0)),
                pl.BlockSpec((1, 1, co), lambda i: (0, 0, 0)),
            ],
            out_specs=pl.BlockSpec((v, tm, co), lambda i: (0, i, 0)),
            compiler_params=pltpu.CompilerParams(
                dimension_semantics=("parallel",),
                vmem_limit_bytes=_VMEM_LIMIT),
        )(xtc, *consts)
        return jnp.transpose(o.reshape(v, nc, t, co), (1, 3, 2, 0))

    if n_chunks == 1:
        return run_chunk(x)
    return jnp.concatenate(
        [run_chunk(x[k * nc:(k + 1) * nc]) for k in range(n_chunks)], axis=0)


# speedup vs baseline: 1.3993x; 1.3993x over previous
from functools import partial

import jax
import jax.numpy as jnp
from jax.experimental import pallas as pl
from jax.experimental.pallas import tpu as pltpu

V_J = 25
_BITS = (1, 2, 4, 8, 16)
_VMEM_LIMIT = 64 * 1024 * 1024


def _fused_kernel(x_ref, m_ref, s1_ref, b1_ref, w_ref, wt_ref, bt_ref,
                  sb_ref, bb_ref, o_ref, *, tm, nsplit=2):
    c = x_ref.shape[-1]
    s_lane = jax.lax.broadcasted_iota(jnp.int32, (1, 1, c), 2) % V_J
    th = tm // nsplit
    rows = V_J * th

    def half(xbf):
        y = xbf
        for b in _BITS:
            y = jnp.where((s_lane & b) != 0,
                          jnp.concatenate([y[b:], y[:b]], axis=0), y)
        xs = y * m_ref[...]
        z = jnp.dot(xs.reshape(rows, c), w_ref[...],
                    preferred_element_type=jnp.float32)
        t = z.reshape(V_J, th, c).astype(jnp.bfloat16) * s1_ref[...] + b1_ref[...]
        for b in _BITS:
            t = jnp.where((s_lane & b) != 0,
                          jnp.concatenate([t[V_J - b:], t[:V_J - b]], axis=0), t)
        g = jnp.maximum(t + xbf, 0)
        u = jnp.dot(g.reshape(rows, c), wt_ref[...],
                    preferred_element_type=jnp.float32)
        u = jnp.maximum(u + bt_ref[...], 0.0)
        return jnp.maximum(
            u.reshape(V_J, th, c) * sb_ref[...] + bb_ref[...] + xbf, 0.0)

    for k in range(nsplit):
        sl = pl.ds(k * th, th)
        o_ref[:, sl, :] = half(x_ref[:, sl, :])


def _pick_tm(nt):
    tm = None
    d = 8
    while d <= min(nt, 256):
        if nt % d == 0:
            tm = d
        d += 8
    return nt if tm is None else tm


def kernel(x, linear_w, linear_b, feature_mask, bn1_scale, bn1_bias,
           bn_a_scale, bn_a_bias, bn_b_scale, bn_b_bias, wt, conv_b,
           idx_in, idx_out):
    n, c, t, v = x.shape
    assert v == V_J
    co = linear_w.shape[1]
    nt = n * t
    tm = _pick_tm(nt)

    mask = jnp.tanh(feature_mask) + 1.0
    b1 = bn1_bias + linear_b * bn1_scale
    wt_f = bn_a_scale.reshape(-1, 1) * wt
    bt_f = bn_a_bias @ wt + conv_b

    jj = jnp.arange(co)[None, :] % V_J
    s1p, b1p = bn1_scale, b1
    for b in _BITS:
        sel = (jj & b) != 0
        s1p = jnp.where(sel, jnp.roll(s1p, -b, axis=0), s1p)
        b1p = jnp.where(sel, jnp.roll(b1p, -b, axis=0), b1p)

    consts = (mask.astype(jnp.bfloat16).reshape(v, 1, c),
              s1p.reshape(v, 1, co).astype(jnp.bfloat16),
              b1p.reshape(v, 1, co).astype(jnp.bfloat16),
              linear_w.astype(jnp.bfloat16), wt_f.astype(jnp.bfloat16),
              bt_f, bn_b_scale.reshape(1, 1, co), bn_b_bias.reshape(1, 1, co))

    xt = jnp.transpose(x, (3, 0, 2, 1)).reshape(v, nt, c).astype(jnp.bfloat16)

    out = pl.pallas_call(
        partial(_fused_kernel, tm=tm),
        out_shape=jax.ShapeDtypeStruct((v, nt, co), jnp.float32),
        grid=(nt // tm,),
        in_specs=[
            pl.BlockSpec((v, tm, c), lambda i: (0, i, 0)),
            pl.BlockSpec((v, 1, c), lambda i: (0, 0, 0)),
            pl.BlockSpec((v, 1, co), lambda i: (0, 0, 0)),
            pl.BlockSpec((v, 1, co), lambda i: (0, 0, 0)),
            pl.BlockSpec((c, co), lambda i: (0, 0)),
            pl.BlockSpec((co, co), lambda i: (0, 0)),
            pl.BlockSpec((1, co), lambda i: (0, 0)),
            pl.BlockSpec((1, 1, co), lambda i: (0, 0, 0)),
            pl.BlockSpec((1, 1, co), lambda i: (0, 0, 0)),
        ],
        out_specs=pl.BlockSpec((v, tm, co), lambda i: (0, i, 0)),
        compiler_params=pltpu.CompilerParams(
            dimension_semantics=("parallel",),
            vmem_limit_bytes=_VMEM_LIMIT),
    )(xt, *consts)

    return jnp.transpose(out.reshape(v, n, t, co), (1, 3, 2, 0))


# speedup vs baseline: 1.4272x; 1.0199x over previous
from functools import partial

import jax
import jax.numpy as jnp
from jax.experimental import pallas as pl
from jax.experimental.pallas import tpu as pltpu

V_J = 25
_BITS = (1, 2, 4, 8, 16)
_VMEM_LIMIT = 64 * 1024 * 1024


def _fused_kernel(x_ref, m_ref, s1_ref, b1_ref, w_ref, wt_ref, bt_ref,
                  sb_ref, bb_ref, o_ref, *, tm, nsplit=2):
    c = x_ref.shape[-1]
    s_lane = jax.lax.broadcasted_iota(jnp.int32, (1, 1, c), 2) % V_J
    th = tm // nsplit
    rows = V_J * th

    def half(xbf):
        y = xbf
        for b in _BITS:
            y = jnp.where((s_lane & b) != 0,
                          jnp.concatenate([y[b:], y[:b]], axis=0), y)
        xs = y * m_ref[...]
        z = jnp.dot(xs.reshape(rows, c), w_ref[...],
                    preferred_element_type=jnp.float32)
        t = z.reshape(V_J, th, c).astype(jnp.bfloat16) * s1_ref[...] + b1_ref[...]
        for b in _BITS:
            t = jnp.where((s_lane & b) != 0,
                          jnp.concatenate([t[V_J - b:], t[:V_J - b]], axis=0), t)
        g = jnp.maximum(t + xbf, 0)
        u = jnp.dot(g.reshape(rows, c), wt_ref[...],
                    preferred_element_type=jnp.float32)
        u = jnp.maximum(u.reshape(V_J, th, c).astype(jnp.bfloat16)
                        + bt_ref[...], 0)
        return jnp.maximum(u * sb_ref[...] + bb_ref[...] + xbf,
                           0).astype(jnp.float32)

    for k in range(nsplit):
        sl = pl.ds(k * th, th)
        o_ref[:, sl, :] = half(x_ref[:, sl, :])


def _pick_tm(nt):
    tm = None
    d = 8
    while d <= min(nt, 256):
        if nt % d == 0:
            tm = d
        d += 8
    return nt if tm is None else tm


def kernel(x, linear_w, linear_b, feature_mask, bn1_scale, bn1_bias,
           bn_a_scale, bn_a_bias, bn_b_scale, bn_b_bias, wt, conv_b,
           idx_in, idx_out):
    n, c, t, v = x.shape
    assert v == V_J
    co = linear_w.shape[1]
    nt = n * t
    tm = _pick_tm(nt)

    mask = jnp.tanh(feature_mask) + 1.0
    b1 = bn1_bias + linear_b * bn1_scale
    wt_f = bn_a_scale.reshape(-1, 1) * wt
    bt_f = bn_a_bias @ wt + conv_b

    jj = jnp.arange(co)[None, :] % V_J
    s1p, b1p = bn1_scale, b1
    for b in _BITS:
        sel = (jj & b) != 0
        s1p = jnp.where(sel, jnp.roll(s1p, -b, axis=0), s1p)
        b1p = jnp.where(sel, jnp.roll(b1p, -b, axis=0), b1p)

    consts = (mask.astype(jnp.bfloat16).reshape(v, 1, c),
              s1p.reshape(v, 1, co).astype(jnp.bfloat16),
              b1p.reshape(v, 1, co).astype(jnp.bfloat16),
              linear_w.astype(jnp.bfloat16), wt_f.astype(jnp.bfloat16),
              bt_f.astype(jnp.bfloat16),
              bn_b_scale.reshape(1, 1, co).astype(jnp.bfloat16),
              bn_b_bias.reshape(1, 1, co).astype(jnp.bfloat16))

    xt = jnp.transpose(x, (3, 0, 2, 1)).reshape(v, nt, c).astype(jnp.bfloat16)

    out = pl.pallas_call(
        partial(_fused_kernel, tm=tm),
        out_shape=jax.ShapeDtypeStruct((v, nt, co), jnp.float32),
        grid=(nt // tm,),
        in_specs=[
            pl.BlockSpec((v, tm, c), lambda i: (0, i, 0)),
            pl.BlockSpec((v, 1, c), lambda i: (0, 0, 0)),
            pl.BlockSpec((v, 1, co), lambda i: (0, 0, 0)),
            pl.BlockSpec((v, 1, co), lambda i: (0, 0, 0)),
            pl.BlockSpec((c, co), lambda i: (0, 0)),
            pl.BlockSpec((co, co), lambda i: (0, 0)),
            pl.BlockSpec((1, co), lambda i: (0, 0)),
            pl.BlockSpec((1, 1, co), lambda i: (0, 0, 0)),
            pl.BlockSpec((1, 1, co), lambda i: (0, 0, 0)),
        ],
        out_specs=pl.BlockSpec((v, tm, co), lambda i: (0, i, 0)),
        compiler_params=pltpu.CompilerParams(
            dimension_semantics=("parallel",),
            vmem_limit_bytes=_VMEM_LIMIT),
    )(xt, *consts)

    return jnp.transpose(out.reshape(v, n, t, co), (1, 3, 2, 0))


# speedup vs baseline: 1.4510x; 1.0166x over previous
from functools import partial

import jax
import jax.numpy as jnp
from jax.experimental import pallas as pl
from jax.experimental.pallas import tpu as pltpu

V_J = 25
_BITS = (1, 2, 4, 8, 16)
_VMEM_LIMIT = 64 * 1024 * 1024


def _fused_kernel(x_ref, m_ref, s1_ref, b1_ref, w_ref, wt_ref, bt_ref,
                  sb_ref, bb_ref, o_ref, *, tm, nsplit=2):
    c = x_ref.shape[-1]
    s_lane = jax.lax.broadcasted_iota(jnp.int32, (1, 1, c), 2) % V_J
    th = tm // nsplit
    rows = V_J * th

    def half(xbf):
        y = xbf
        for b in _BITS:
            y = jnp.where((s_lane & b) != 0,
                          jnp.concatenate([y[b:], y[:b]], axis=0), y)
        xs = y * m_ref[...]
        z = jnp.dot(xs.reshape(rows, c), w_ref[...],
                    preferred_element_type=jnp.float32)
        t = z.reshape(V_J, th, c).astype(jnp.bfloat16) * s1_ref[...] + b1_ref[...]
        for b in _BITS:
            t = jnp.where((s_lane & b) != 0,
                          jnp.concatenate([t[V_J - b:], t[:V_J - b]], axis=0), t)
        g = jnp.maximum(t + xbf, 0)
        u = jnp.dot(g.reshape(rows, c), wt_ref[...],
                    preferred_element_type=jnp.float32)
        u = jnp.maximum(u.reshape(V_J, th, c).astype(jnp.bfloat16)
                        + bt_ref[...], 0)
        return jnp.maximum(u * sb_ref[...] + bb_ref[...] + xbf,
                           0).astype(jnp.float32)

    for k in range(nsplit):
        sl = pl.ds(k * th, th)
        o_ref[:, sl, :] = half(x_ref[:, sl, :].astype(jnp.bfloat16))


def _pick_tm(nt):
    tm = None
    d = 8
    while d <= min(nt, 256):
        if nt % d == 0:
            tm = d
        d += 8
    return nt if tm is None else tm


def kernel(x, linear_w, linear_b, feature_mask, bn1_scale, bn1_bias,
           bn_a_scale, bn_a_bias, bn_b_scale, bn_b_bias, wt, conv_b,
           idx_in, idx_out):
    n, c, t, v = x.shape
    assert v == V_J
    co = linear_w.shape[1]
    nt = n * t
    tm = _pick_tm(nt)

    mask = jnp.tanh(feature_mask) + 1.0
    b1 = bn1_bias + linear_b * bn1_scale
    wt_f = bn_a_scale.reshape(-1, 1) * wt
    bt_f = bn_a_bias @ wt + conv_b

    jj = jnp.arange(co)[None, :] % V_J
    s1p, b1p = bn1_scale, b1
    for b in _BITS:
        sel = (jj & b) != 0
        s1p = jnp.where(sel, jnp.roll(s1p, -b, axis=0), s1p)
        b1p = jnp.where(sel, jnp.roll(b1p, -b, axis=0), b1p)

    consts = (mask.astype(jnp.bfloat16).reshape(v, 1, c),
              s1p.reshape(v, 1, co).astype(jnp.bfloat16),
              b1p.reshape(v, 1, co).astype(jnp.bfloat16),
              linear_w.astype(jnp.bfloat16), wt_f.astype(jnp.bfloat16),
              bt_f.astype(jnp.bfloat16),
              bn_b_scale.reshape(1, 1, co).astype(jnp.bfloat16),
              bn_b_bias.reshape(1, 1, co).astype(jnp.bfloat16))

    xt = jnp.transpose(x, (3, 0, 2, 1)).reshape(v, nt, c)

    out = pl.pallas_call(
        partial(_fused_kernel, tm=tm),
        out_shape=jax.ShapeDtypeStruct((v, nt, co), jnp.float32),
        grid=(nt // tm,),
        in_specs=[
            pl.BlockSpec((v, tm, c), lambda i: (0, i, 0)),
            pl.BlockSpec((v, 1, c), lambda i: (0, 0, 0)),
            pl.BlockSpec((v, 1, co), lambda i: (0, 0, 0)),
            pl.BlockSpec((v, 1, co), lambda i: (0, 0, 0)),
            pl.BlockSpec((c, co), lambda i: (0, 0)),
            pl.BlockSpec((co, co), lambda i: (0, 0)),
            pl.BlockSpec((1, co), lambda i: (0, 0)),
            pl.BlockSpec((1, 1, co), lambda i: (0, 0, 0)),
            pl.BlockSpec((1, 1, co), lambda i: (0, 0, 0)),
        ],
        out_specs=pl.BlockSpec((v, tm, co), lambda i: (0, i, 0)),
        compiler_params=pltpu.CompilerParams(
            dimension_semantics=("parallel",),
            vmem_limit_bytes=_VMEM_LIMIT),
    )(xt, *consts)

    return jnp.transpose(out.reshape(v, n, t, co), (1, 3, 2, 0))
